# Initial kernel scaffold; baseline (speedup 1.0000x reference)
import jax
import jax.numpy as jnp
from jax.experimental import pallas as pl


def kernel(input_ids, position_ids, subword_table, positional_table):
    raise NotImplementedError("write your pallas kernel here")


# speedup vs baseline: 5.1236x; 5.1236x over previous
import functools

import jax
import jax.numpy as jnp
from jax import lax
from jax.experimental import pallas as pl
from jax.experimental.pallas import tpu as pltpu
from jax.experimental.pallas import tpu_sc as plsc

H = 128
NC = 2
NS = 16
NW = NC * NS
LANES = 16
W = 128


def _sc_embed(ids, pids, subword_table, positional_table, n):
    bpw = n // NW
    steps = bpw // W
    mesh = plsc.VectorSubcoreMesh(core_axis_name="c", subcore_axis_name="s")

    @functools.partial(
        pl.kernel,
        mesh=mesh,
        out_type=jax.ShapeDtypeStruct((n, H), jnp.float32),
        scratch_types=[
            pltpu.VMEM((W,), jnp.int32),
            pltpu.VMEM((W,), jnp.int32),
            pltpu.VMEM((W, H), jnp.float32),
            pltpu.VMEM((W, H), jnp.float32),
            pltpu.SemaphoreType.DMA,
            pltpu.SemaphoreType.DMA,
        ],
    )
    def k(sub_hbm, pos_hbm, ids_hbm, pids_hbm, out_hbm,
          ids_v, pids_v, rows_v, prow_v, sem1, sem2):
        wid = lax.axis_index("s") * NC + lax.axis_index("c")
        base = wid * bpw

        @pl.loop(0, steps)
        def _(step):
            off = base + step * W
            pltpu.sync_copy(ids_hbm.at[pl.ds(off, W)], ids_v)
            pltpu.sync_copy(pids_hbm.at[pl.ds(off, W)], pids_v)
            c1 = pltpu.async_copy(sub_hbm.at[ids_v], rows_v, sem1)
            c2 = pltpu.async_copy(pos_hbm.at[pids_v], prow_v, sem2)
            c1.wait()
            c2.wait()

            @pl.loop(0, W)
            def _(r):
                for h in range(0, H, LANES):
                    sl = pl.ds(h, LANES)
                    rows_v[r, sl] = rows_v[r, sl] + prow_v[r, sl]

            pltpu.sync_copy(rows_v, out_hbm.at[pl.ds(off, W)])

    return k(subword_table, positional_table, ids, pids)


def kernel(input_ids, position_ids, subword_table, positional_table):
    b, s = input_ids.shape
    n = b * s
    out = _sc_embed(
        input_ids.reshape(n),
        position_ids.reshape(n),
        subword_table,
        positional_table,
        n,
    )
    return out.reshape(b, s, H)
